# baseline (speedup 1.0000x reference)
import jax
import jax.numpy as jnp
from jax import lax
from jax.experimental import pallas as pl

_DIM_2 = 128
_BB = 256
_WIN = 5504
_BIGF = 3.0e38


def _round_bf16(v):
    u = lax.bitcast_convert_type(v, jnp.uint32)
    r = (u + jnp.uint32(0x7FFF) + ((u >> 16) & jnp.uint32(1))) & jnp.uint32(0xFFFF0000)
    return lax.bitcast_convert_type(r, jnp.float32)


def _win_min_argmin(d2, iota, lo, hi):
    blk = d2[:, lo:hi]
    m = jnp.min(blk, axis=1, keepdims=True)
    bi = jnp.min(jnp.where(blk == m, iota[:, lo:hi], _BIGF),
                 axis=1, keepdims=True)
    return m, bi


def _bmu_body(xm2_ref, w_ref, xsq_ref, wsq_ref, iota_ref, out_ref):
    mm = lax.dot_general(
        xm2_ref[...], w_ref[...],
        dimension_numbers=(((1,), (1,)), ((), ())),
        preferred_element_type=jnp.float32,
    )
    d2 = (xsq_ref[...] + wsq_ref[...]) + mm

    iota = iota_ref[...]
    k = d2.shape[1]
    bounds = list(range(0, k, _WIN)) + [k]
    m0, i0 = _win_min_argmin(d2, iota, bounds[0], bounds[1])
    cur_v = _round_bf16(jnp.sqrt(jnp.maximum(m0, 0.0)))
    cur_i = i0
    for w in range(1, len(bounds) - 1):
        mw, iw = _win_min_argmin(d2, iota, bounds[w], bounds[w + 1])
        dw = jnp.sqrt(jnp.maximum(mw, 0.0))
        take = dw < cur_v
        cur_v = jnp.where(take, _round_bf16(dw), cur_v)
        cur_i = jnp.where(take, iw, cur_i)

    idx = cur_i.astype(jnp.int32)
    out_ref[...] = jnp.concatenate([idx >> 7, idx & (_DIM_2 - 1)], axis=1)


@jax.jit
def kernel(x, weights):
    b, d = x.shape
    k, _ = weights.shape
    nbb = b // _BB

    x_sq = jnp.sum(x * x, axis=1, keepdims=True)
    w_sq = jnp.sum(weights * weights, axis=1)[None, :]
    xm2 = (-2.0 * x).astype(jnp.bfloat16)
    w16 = weights.astype(jnp.bfloat16)
    iota_k = jnp.arange(k, dtype=jnp.float32)[None, :]

    return pl.pallas_call(
        _bmu_body,
        grid=(nbb,),
        in_specs=[
            pl.BlockSpec((_BB, d), lambda ib: (ib, 0)),
            pl.BlockSpec((k, d), lambda ib: (0, 0)),
            pl.BlockSpec((_BB, 1), lambda ib: (ib, 0)),
            pl.BlockSpec((1, k), lambda ib: (0, 0)),
            pl.BlockSpec((1, k), lambda ib: (0, 0)),
        ],
        out_specs=pl.BlockSpec((_BB, 2), lambda ib: (ib, 0)),
        out_shape=jax.ShapeDtypeStruct((b, 2), jnp.int32),
    )(xm2, w16, x_sq, w_sq, iota_k)
